# Initial kernel scaffold; baseline (speedup 1.0000x reference)
import jax
import jax.numpy as jnp
from jax.experimental import pallas as pl


def kernel(x, edge_index, W0, b0, W1, b1, W2, b2, H1w, H1b, H2w, H2b, H3w, H3b):
    raise NotImplementedError("write your pallas kernel here")


# speedup vs baseline: 7.2406x; 7.2406x over previous
import jax
import jax.numpy as jnp
from jax import lax
from jax.experimental import pallas as pl
from jax.experimental.pallas import tpu as pltpu, tpu_sc as plsc

N = 50000
E = 800000
IN_DIM = 300
EMB = 64
HALF = 32

NSC = 2
NSUB = 16
NPT = N // NSUB

_MM = dict(preferred_element_type=jnp.float32, precision=lax.Precision.HIGHEST)

DEG_C = 40
DEG_EPS = E // NSC
DEG_EPT = DEG_EPS // NSUB
DEG_NCH = DEG_EPT // DEG_C


def _deg_body(dst_hbm, ones_hbm, zeros_hbm, p0_hbm, p1_hbm,
              acc_sp, dst_v, ones_v):
    c = lax.axis_index("c")
    s = lax.axis_index("s")
    row = pl.ds(s * NPT, NPT)
    pltpu.sync_copy(zeros_hbm.at[row], acc_sp.at[row])
    pltpu.sync_copy(ones_hbm, ones_v)
    plsc.subcore_barrier()
    base = c * DEG_EPS + s * DEG_EPT

    def body(k, carry):
        off = base + k * DEG_C
        pltpu.sync_copy(dst_hbm.at[pl.ds(off, DEG_C)], dst_v)
        pltpu.sync_copy(ones_v, acc_sp.at[dst_v], add=True)
        return carry

    lax.fori_loop(0, DEG_NCH, body, 0)
    plsc.subcore_barrier()

    @pl.when(c == 0)
    def _():
        pltpu.sync_copy(acc_sp.at[row], p0_hbm.at[row])

    @pl.when(c == 1)
    def _():
        pltpu.sync_copy(acc_sp.at[row], p1_hbm.at[row])


import functools


@functools.cache
def _get_deg():
    return pl.kernel(
        _deg_body,
        out_type=[
            jax.ShapeDtypeStruct((N, 8), jnp.float32),
            jax.ShapeDtypeStruct((N, 8), jnp.float32),
        ],
        mesh=plsc.VectorSubcoreMesh(
            core_axis_name="c", subcore_axis_name="s",
            num_cores=NSC, num_subcores=NSUB),
        scratch_types=[
            pltpu.VMEM_SHARED((N, 8), jnp.float32),
            pltpu.VMEM((DEG_C,), jnp.int32),
            pltpu.VMEM((DEG_C, 8), jnp.float32),
        ],
        compiler_params=pltpu.CompilerParams(use_tc_tiling_on_sc=False),
    )

PROP_C = 80
PROP_EPT = E // NSUB
PROP_NCH = PROP_EPT // PROP_C


def _prop_body(g0_hbm, g1_hbm, src_hbm, dst_hbm, a0_hbm, a1_hbm,
               acc_sp, src_v, dst_v, rows_v, sem):
    c = lax.axis_index("c")
    s = lax.axis_index("s")
    row = pl.ds(s * NPT, NPT)

    def run(g_hbm, a_hbm):
        pltpu.sync_copy(g_hbm.at[row], acc_sp.at[row])
        plsc.subcore_barrier()

        def body(k, carry):
            off = s * PROP_EPT + k * PROP_C
            pltpu.sync_copy(src_hbm.at[pl.ds(off, PROP_C)], src_v)
            pltpu.sync_copy(dst_hbm.at[pl.ds(off, PROP_C)], dst_v)
            pltpu.async_copy(g_hbm.at[src_v], rows_v, sem).wait()
            pltpu.sync_copy(rows_v, acc_sp.at[dst_v], add=True)
            return carry

        lax.fori_loop(0, PROP_NCH, body, 0)
        plsc.subcore_barrier()
        pltpu.sync_copy(acc_sp.at[row], a_hbm.at[row])

    @pl.when(c == 0)
    def _():
        run(g0_hbm, a0_hbm)

    @pl.when(c == 1)
    def _():
        run(g1_hbm, a1_hbm)


@functools.cache
def _get_prop():
    return pl.kernel(
        _prop_body,
        out_type=[
            jax.ShapeDtypeStruct((N, HALF), jnp.float32),
            jax.ShapeDtypeStruct((N, HALF), jnp.float32),
        ],
        mesh=plsc.VectorSubcoreMesh(
            core_axis_name="c", subcore_axis_name="s",
            num_cores=NSC, num_subcores=NSUB),
        scratch_types=[
            pltpu.VMEM_SHARED((N, HALF), jnp.float32),
            pltpu.VMEM((PROP_C,), jnp.int32),
            pltpu.VMEM((PROP_C,), jnp.int32),
            pltpu.VMEM((PROP_C, HALF), jnp.float32),
            pltpu.SemaphoreType.DMA,
        ],
        compiler_params=pltpu.CompilerParams(use_tc_tiling_on_sc=False),
    )

BLK = 2000
GRID = N // BLK


def _tc0_body(x_ref, w_ref, p0_ref, p1_ref, g0_ref, g1_ref, dinv_ref):
    deg = 1.0 + p0_ref[:, 0:1] + p1_ref[:, 0:1]
    dinv = lax.rsqrt(deg)
    h = jnp.dot(x_ref[...], w_ref[...], **_MM)
    gg = h * dinv
    g0_ref[...] = gg[:, :HALF]
    g1_ref[...] = gg[:, HALF:]
    dinv_ref[...] = dinv


_tc0 = pl.pallas_call(
    _tc0_body,
    grid=(GRID,),
    in_specs=[
        pl.BlockSpec((BLK, IN_DIM), lambda i: (i, 0)),
        pl.BlockSpec((IN_DIM, EMB), lambda i: (0, 0)),
        pl.BlockSpec((BLK, 8), lambda i: (i, 0)),
        pl.BlockSpec((BLK, 8), lambda i: (i, 0)),
    ],
    out_specs=[
        pl.BlockSpec((BLK, HALF), lambda i: (i, 0)),
        pl.BlockSpec((BLK, HALF), lambda i: (i, 0)),
        pl.BlockSpec((BLK, 1), lambda i: (i, 0)),
    ],
    out_shape=[
        jax.ShapeDtypeStruct((N, HALF), jnp.float32),
        jax.ShapeDtypeStruct((N, HALF), jnp.float32),
        jax.ShapeDtypeStruct((N, 1), jnp.float32),
    ],
)


def _tc_mid_body(a0_ref, a1_ref, dinv_ref, b_ref, w_ref, g0_ref, g1_ref):
    accf = jnp.concatenate([a0_ref[...], a1_ref[...]], axis=1)
    dinv = dinv_ref[...]
    act = jnp.maximum(accf * dinv + b_ref[...], 0.0)
    gg = jnp.dot(act, w_ref[...], **_MM) * dinv
    g0_ref[...] = gg[:, :HALF]
    g1_ref[...] = gg[:, HALF:]


_tc_mid = pl.pallas_call(
    _tc_mid_body,
    grid=(GRID,),
    in_specs=[
        pl.BlockSpec((BLK, HALF), lambda i: (i, 0)),
        pl.BlockSpec((BLK, HALF), lambda i: (i, 0)),
        pl.BlockSpec((BLK, 1), lambda i: (i, 0)),
        pl.BlockSpec((1, EMB), lambda i: (0, 0)),
        pl.BlockSpec((EMB, EMB), lambda i: (0, 0)),
    ],
    out_specs=[
        pl.BlockSpec((BLK, HALF), lambda i: (i, 0)),
        pl.BlockSpec((BLK, HALF), lambda i: (i, 0)),
    ],
    out_shape=[
        jax.ShapeDtypeStruct((N, HALF), jnp.float32),
        jax.ShapeDtypeStruct((N, HALF), jnp.float32),
    ],
)


def _tc3_body(a0_ref, a1_ref, dinv_ref, b2_ref, h1w_ref, h1b_ref,
              h2w_ref, h2b_ref, h3w_ref, h3b_ref, out_ref):
    accf = jnp.concatenate([a0_ref[...], a1_ref[...]], axis=1)
    h3 = accf * dinv_ref[...] + b2_ref[...]
    t = jnp.maximum(jnp.dot(h3, h1w_ref[...], **_MM) + h1b_ref[...], 0.0)
    t = jnp.maximum(jnp.dot(t, h2w_ref[...], **_MM) + h2b_ref[...], 0.0)
    out_ref[...] = jnp.dot(t, h3w_ref[...], **_MM) + h3b_ref[...]


_tc3 = pl.pallas_call(
    _tc3_body,
    grid=(GRID,),
    in_specs=[
        pl.BlockSpec((BLK, HALF), lambda i: (i, 0)),
        pl.BlockSpec((BLK, HALF), lambda i: (i, 0)),
        pl.BlockSpec((BLK, 1), lambda i: (i, 0)),
        pl.BlockSpec((1, EMB), lambda i: (0, 0)),
        pl.BlockSpec((EMB, EMB), lambda i: (0, 0)),
        pl.BlockSpec((1, EMB), lambda i: (0, 0)),
        pl.BlockSpec((EMB, EMB), lambda i: (0, 0)),
        pl.BlockSpec((1, EMB), lambda i: (0, 0)),
        pl.BlockSpec((EMB, EMB), lambda i: (0, 0)),
        pl.BlockSpec((1, EMB), lambda i: (0, 0)),
    ],
    out_specs=[pl.BlockSpec((BLK, EMB), lambda i: (i, 0))],
    out_shape=[jax.ShapeDtypeStruct((N, EMB), jnp.float32)],
)


def kernel(x, edge_index, W0, b0, W1, b1, W2, b2, H1w, H1b, H2w, H2b, H3w, H3b):
    src = edge_index[0]
    dst = edge_index[1]
    ones = jnp.ones((DEG_C, 8), jnp.float32)
    zeros = jnp.zeros((N, 8), jnp.float32)

    p0, p1 = _get_deg()(dst, ones, zeros)
    _prop = _get_prop()
    g0, g1, dinv = _tc0(x, W0, p0, p1)
    a0, a1 = _prop(g0, g1, src, dst)
    g0, g1 = _tc_mid(a0, a1, dinv, b0.reshape(1, EMB), W1)
    a0, a1 = _prop(g0, g1, src, dst)
    g0, g1 = _tc_mid(a0, a1, dinv, b1.reshape(1, EMB), W2)
    a0, a1 = _prop(g0, g1, src, dst)
    (out,) = _tc3(a0, a1, dinv, b2.reshape(1, EMB),
                  H1w, H1b.reshape(1, EMB), H2w, H2b.reshape(1, EMB),
                  H3w, H3b.reshape(1, EMB))
    return out


# speedup vs baseline: 19.7409x; 2.7264x over previous
import jax
import jax.numpy as jnp
from jax import lax
from jax.experimental import pallas as pl
from jax.experimental.pallas import tpu as pltpu, tpu_sc as plsc

N = 50000
E = 800000
IN_DIM = 300
EMB = 64
HALF = 32

NSC = 2
NSUB = 16
NPT = N // NSUB

_MM = dict(preferred_element_type=jnp.float32, precision=lax.Precision.HIGHEST)

import functools

PC = 100
PROWS = E // PC
DROWS = PROWS // (NSC * NSUB)


def _deg_body(dst2_hbm, ones_hbm, zeros_hbm, p0_hbm, p1_hbm,
              acc_sp, dsta, ones_v, sem0, sem1):
    c = lax.axis_index("c")
    s = lax.axis_index("s")
    row = pl.ds(s * NPT, NPT)
    pltpu.sync_copy(dst2_hbm.at[pl.ds((c * NSUB + s) * DROWS, DROWS)], dsta)
    pltpu.sync_copy(zeros_hbm.at[row], acc_sp.at[row])
    pltpu.sync_copy(ones_hbm, ones_v)
    plsc.subcore_barrier()

    pltpu.async_copy(ones_v, acc_sp.at[dsta.at[0]], sem0, add=True)
    pltpu.async_copy(ones_v, acc_sp.at[dsta.at[1]], sem1, add=True)

    def pair(g, carry):
        k0 = g * 2
        k1 = k0 + 1
        pltpu.make_async_copy(ones_v, acc_sp.at[dsta.at[k0 - 2]], sem0).wait()
        pltpu.async_copy(ones_v, acc_sp.at[dsta.at[k0]], sem0, add=True)
        pltpu.make_async_copy(ones_v, acc_sp.at[dsta.at[k1 - 2]], sem1).wait()
        pltpu.async_copy(ones_v, acc_sp.at[dsta.at[k1]], sem1, add=True)
        return carry

    lax.fori_loop(1, DROWS // 2, pair, 0)
    pltpu.make_async_copy(ones_v, acc_sp.at[dsta.at[DROWS - 2]], sem0).wait()
    pltpu.make_async_copy(ones_v, acc_sp.at[dsta.at[DROWS - 1]], sem1).wait()
    plsc.subcore_barrier()

    @pl.when(c == 0)
    def _():
        pltpu.sync_copy(acc_sp.at[row], p0_hbm.at[row])

    @pl.when(c == 1)
    def _():
        pltpu.sync_copy(acc_sp.at[row], p1_hbm.at[row])


@functools.cache
def _get_deg():
    return pl.kernel(
        _deg_body,
        out_type=[
            jax.ShapeDtypeStruct((N, 8), jnp.float32),
            jax.ShapeDtypeStruct((N, 8), jnp.float32),
        ],
        mesh=plsc.VectorSubcoreMesh(
            core_axis_name="c", subcore_axis_name="s",
            num_cores=NSC, num_subcores=NSUB),
        scratch_types=[
            pltpu.VMEM_SHARED((N, 8), jnp.float32),
            pltpu.VMEM((DROWS, PC), jnp.int32),
            pltpu.VMEM((PC, 8), jnp.float32),
            pltpu.SemaphoreType.DMA,
            pltpu.SemaphoreType.DMA,
        ],
        compiler_params=pltpu.CompilerParams(use_tc_tiling_on_sc=False),
    )

TROWS = PROWS // NSUB
IB = 10
NB = TROWS // IB


def _prop_body(g0_hbm, g1_hbm, src2_hbm, dst2_hbm, a0_hbm, a1_hbm,
               acc_sp, srcb, dstb, rows,
               sem_i0, sem_i1, sem_g0, sem_g1):
    c = lax.axis_index("c")
    s = lax.axis_index("s")
    row = pl.ds(s * NPT, NPT)
    sem_i = (sem_i0, sem_i1)
    sem_g = (sem_g0, sem_g1)

    def idx_copies(b, slot, sem):
        off = pl.ds(s * TROWS + b * IB, IB)
        return (pltpu.make_async_copy(src2_hbm.at[off], srcb.at[slot], sem),
                pltpu.make_async_copy(dst2_hbm.at[off], dstb.at[slot], sem))

    def issue_idx(b, slot, sem):
        for d in idx_copies(b, slot, sem):
            d.start()

    def wait_idx(b, slot, sem):
        for d in idx_copies(b, slot, sem):
            d.wait()

    def run(g_hbm, a_hbm):
        pltpu.sync_copy(g_hbm.at[row], acc_sp.at[row])
        issue_idx(0, 0, sem_i[0])
        plsc.subcore_barrier()

        def superblock(q, carry):
            for half in (0, 1):
                b = q * 2 + half
                wait_idx(b, half, sem_i[half])

                @pl.when(b < NB - 1)
                def _():
                    issue_idx(b + 1, 1 - half, sem_i[1 - half])

                sb = srcb.at[half]
                db = dstb.at[half]
                pltpu.async_copy(g_hbm.at[sb.at[0]], rows.at[0], sem_g0)

                def pair(g, carry2):
                    j0 = g * 2
                    j1 = j0 + 1
                    pltpu.async_copy(g_hbm.at[sb.at[j1]], rows.at[1], sem_g1)
                    pltpu.make_async_copy(
                        g_hbm.at[sb.at[j0]], rows.at[0], sem_g0).wait()
                    pltpu.sync_copy(rows.at[0], acc_sp.at[db.at[j0]], add=True)

                    @pl.when(g < IB // 2 - 1)
                    def _():
                        pltpu.async_copy(
                            g_hbm.at[sb.at[j0 + 2]], rows.at[0], sem_g0)

                    pltpu.make_async_copy(
                        g_hbm.at[sb.at[j1]], rows.at[1], sem_g1).wait()
                    pltpu.sync_copy(rows.at[1], acc_sp.at[db.at[j1]], add=True)
                    return carry2

                lax.fori_loop(0, IB // 2, pair, 0)
            return carry

        lax.fori_loop(0, NB // 2, superblock, 0)
        plsc.subcore_barrier()
        pltpu.sync_copy(acc_sp.at[row], a_hbm.at[row])

    @pl.when(c == 0)
    def _():
        run(g0_hbm, a0_hbm)

    @pl.when(c == 1)
    def _():
        run(g1_hbm, a1_hbm)


@functools.cache
def _get_prop():
    return pl.kernel(
        _prop_body,
        out_type=[
            jax.ShapeDtypeStruct((N, HALF), jnp.float32),
            jax.ShapeDtypeStruct((N, HALF), jnp.float32),
        ],
        mesh=plsc.VectorSubcoreMesh(
            core_axis_name="c", subcore_axis_name="s",
            num_cores=NSC, num_subcores=NSUB),
        scratch_types=[
            pltpu.VMEM_SHARED((N, HALF), jnp.float32),
            pltpu.VMEM((2, IB, PC), jnp.int32),
            pltpu.VMEM((2, IB, PC), jnp.int32),
            pltpu.VMEM((2, PC, HALF), jnp.float32),
            pltpu.SemaphoreType.DMA,
            pltpu.SemaphoreType.DMA,
            pltpu.SemaphoreType.DMA,
            pltpu.SemaphoreType.DMA,
        ],
        compiler_params=pltpu.CompilerParams(use_tc_tiling_on_sc=False),
    )

BLK = 2000
GRID = N // BLK


def _tc0_body(x_ref, w_ref, p0_ref, p1_ref, g0_ref, g1_ref, dinv_ref):
    deg = 1.0 + p0_ref[:, 0:1] + p1_ref[:, 0:1]
    dinv = lax.rsqrt(deg)
    h = jnp.dot(x_ref[...], w_ref[...], **_MM)
    gg = h * dinv
    g0_ref[...] = gg[:, :HALF]
    g1_ref[...] = gg[:, HALF:]
    dinv_ref[...] = dinv


_tc0 = pl.pallas_call(
    _tc0_body,
    grid=(GRID,),
    in_specs=[
        pl.BlockSpec((BLK, IN_DIM), lambda i: (i, 0)),
        pl.BlockSpec((IN_DIM, EMB), lambda i: (0, 0)),
        pl.BlockSpec((BLK, 8), lambda i: (i, 0)),
        pl.BlockSpec((BLK, 8), lambda i: (i, 0)),
    ],
    out_specs=[
        pl.BlockSpec((BLK, HALF), lambda i: (i, 0)),
        pl.BlockSpec((BLK, HALF), lambda i: (i, 0)),
        pl.BlockSpec((BLK, 1), lambda i: (i, 0)),
    ],
    out_shape=[
        jax.ShapeDtypeStruct((N, HALF), jnp.float32),
        jax.ShapeDtypeStruct((N, HALF), jnp.float32),
        jax.ShapeDtypeStruct((N, 1), jnp.float32),
    ],
)


def _tc_mid_body(a0_ref, a1_ref, dinv_ref, b_ref, w_ref, g0_ref, g1_ref):
    accf = jnp.concatenate([a0_ref[...], a1_ref[...]], axis=1)
    dinv = dinv_ref[...]
    act = jnp.maximum(accf * dinv + b_ref[...], 0.0)
    gg = jnp.dot(act, w_ref[...], **_MM) * dinv
    g0_ref[...] = gg[:, :HALF]
    g1_ref[...] = gg[:, HALF:]


_tc_mid = pl.pallas_call(
    _tc_mid_body,
    grid=(GRID,),
    in_specs=[
        pl.BlockSpec((BLK, HALF), lambda i: (i, 0)),
        pl.BlockSpec((BLK, HALF), lambda i: (i, 0)),
        pl.BlockSpec((BLK, 1), lambda i: (i, 0)),
        pl.BlockSpec((1, EMB), lambda i: (0, 0)),
        pl.BlockSpec((EMB, EMB), lambda i: (0, 0)),
    ],
    out_specs=[
        pl.BlockSpec((BLK, HALF), lambda i: (i, 0)),
        pl.BlockSpec((BLK, HALF), lambda i: (i, 0)),
    ],
    out_shape=[
        jax.ShapeDtypeStruct((N, HALF), jnp.float32),
        jax.ShapeDtypeStruct((N, HALF), jnp.float32),
    ],
)


def _tc3_body(a0_ref, a1_ref, dinv_ref, b2_ref, h1w_ref, h1b_ref,
              h2w_ref, h2b_ref, h3w_ref, h3b_ref, out_ref):
    accf = jnp.concatenate([a0_ref[...], a1_ref[...]], axis=1)
    h3 = accf * dinv_ref[...] + b2_ref[...]
    t = jnp.maximum(jnp.dot(h3, h1w_ref[...], **_MM) + h1b_ref[...], 0.0)
    t = jnp.maximum(jnp.dot(t, h2w_ref[...], **_MM) + h2b_ref[...], 0.0)
    out_ref[...] = jnp.dot(t, h3w_ref[...], **_MM) + h3b_ref[...]


_tc3 = pl.pallas_call(
    _tc3_body,
    grid=(GRID,),
    in_specs=[
        pl.BlockSpec((BLK, HALF), lambda i: (i, 0)),
        pl.BlockSpec((BLK, HALF), lambda i: (i, 0)),
        pl.BlockSpec((BLK, 1), lambda i: (i, 0)),
        pl.BlockSpec((1, EMB), lambda i: (0, 0)),
        pl.BlockSpec((EMB, EMB), lambda i: (0, 0)),
        pl.BlockSpec((1, EMB), lambda i: (0, 0)),
        pl.BlockSpec((EMB, EMB), lambda i: (0, 0)),
        pl.BlockSpec((1, EMB), lambda i: (0, 0)),
        pl.BlockSpec((EMB, EMB), lambda i: (0, 0)),
        pl.BlockSpec((1, EMB), lambda i: (0, 0)),
    ],
    out_specs=[pl.BlockSpec((BLK, EMB), lambda i: (i, 0))],
    out_shape=[jax.ShapeDtypeStruct((N, EMB), jnp.float32)],
)


def kernel(x, edge_index, W0, b0, W1, b1, W2, b2, H1w, H1b, H2w, H2b, H3w, H3b):
    src2 = edge_index[0].reshape(PROWS, PC)
    dst2 = edge_index[1].reshape(PROWS, PC)
    ones = jnp.ones((PC, 8), jnp.float32)
    zeros = jnp.zeros((N, 8), jnp.float32)

    p0, p1 = _get_deg()(dst2, ones, zeros)
    _prop = _get_prop()
    g0, g1, dinv = _tc0(x, W0, p0, p1)
    a0, a1 = _prop(g0, g1, src2, dst2)
    g0, g1 = _tc_mid(a0, a1, dinv, b0.reshape(1, EMB), W1)
    a0, a1 = _prop(g0, g1, src2, dst2)
    g0, g1 = _tc_mid(a0, a1, dinv, b1.reshape(1, EMB), W2)
    a0, a1 = _prop(g0, g1, src2, dst2)
    (out,) = _tc3(a0, a1, dinv, b2.reshape(1, EMB),
                  H1w, H1b.reshape(1, EMB), H2w, H2b.reshape(1, EMB),
                  H3w, H3b.reshape(1, EMB))
    return out


# speedup vs baseline: 20.5329x; 1.0401x over previous
import functools

import jax
import jax.numpy as jnp
from jax import lax
from jax.experimental import pallas as pl
from jax.experimental.pallas import tpu as pltpu, tpu_sc as plsc

N = 50000
E = 800000
IN_DIM = 300
EMB = 64
HALF = 32

NSC = 2
NSUB = 16
NPT = N // NSUB

_MM = dict(preferred_element_type=jnp.float32, precision=lax.Precision.HIGHEST)

PC = 100
PROWS = E // PC
DROWS = PROWS // (NSC * NSUB)


def _deg_body(dst2_hbm, ones_hbm, zeros_hbm, p_hbm,
              acc_sp, dsta, ones_v, sem0, sem1):
    c = lax.axis_index("c")
    s = lax.axis_index("s")
    row = pl.ds(s * NPT, NPT)
    pltpu.sync_copy(dst2_hbm.at[pl.ds((c * NSUB + s) * DROWS, DROWS)], dsta)
    pltpu.sync_copy(zeros_hbm.at[row], acc_sp.at[row])
    pltpu.sync_copy(ones_hbm, ones_v)
    plsc.subcore_barrier()

    pltpu.async_copy(ones_v, acc_sp.at[dsta.at[0]], sem0, add=True)
    pltpu.async_copy(ones_v, acc_sp.at[dsta.at[1]], sem1, add=True)

    def pair(g, carry):
        k0 = g * 2
        k1 = k0 + 1
        pltpu.make_async_copy(ones_v, acc_sp.at[dsta.at[k0 - 2]], sem0).wait()
        pltpu.async_copy(ones_v, acc_sp.at[dsta.at[k0]], sem0, add=True)
        pltpu.make_async_copy(ones_v, acc_sp.at[dsta.at[k1 - 2]], sem1).wait()
        pltpu.async_copy(ones_v, acc_sp.at[dsta.at[k1]], sem1, add=True)
        return carry

    lax.fori_loop(1, DROWS // 2, pair, 0)
    pltpu.make_async_copy(ones_v, acc_sp.at[dsta.at[DROWS - 2]], sem0).wait()
    pltpu.make_async_copy(ones_v, acc_sp.at[dsta.at[DROWS - 1]], sem1).wait()
    plsc.subcore_barrier()

    @pl.when(c == 0)
    def _():
        pltpu.sync_copy(acc_sp.at[row], p_hbm.at[row, pl.ds(0, 8)])

    @pl.when(c == 1)
    def _():
        pltpu.sync_copy(acc_sp.at[row], p_hbm.at[row, pl.ds(8, 8)])


@functools.cache
def _get_deg():
    return pl.kernel(
        _deg_body,
        out_type=[
            jax.ShapeDtypeStruct((N, 16), jnp.float32),
        ],
        mesh=plsc.VectorSubcoreMesh(
            core_axis_name="c", subcore_axis_name="s",
            num_cores=NSC, num_subcores=NSUB),
        scratch_types=[
            pltpu.VMEM_SHARED((N, 8), jnp.float32),
            pltpu.VMEM((DROWS, PC), jnp.int32),
            pltpu.VMEM((PC, 8), jnp.float32),
            pltpu.SemaphoreType.DMA,
            pltpu.SemaphoreType.DMA,
        ],
        compiler_params=pltpu.CompilerParams(use_tc_tiling_on_sc=False),
    )

TROWS = PROWS // NSUB
IB = 10
NB = TROWS // IB


def _prop_body(g0_hbm, g1_hbm, src2_hbm, dst2_hbm, a_hbm,
               acc_sp, srcb, dstb, rows,
               sem_i0, sem_i1, sem_g0, sem_g1):
    c = lax.axis_index("c")
    s = lax.axis_index("s")
    row = pl.ds(s * NPT, NPT)
    sem_i = (sem_i0, sem_i1)

    def idx_copies(b, slot, sem):
        off = pl.ds(s * TROWS + b * IB, IB)
        return (pltpu.make_async_copy(src2_hbm.at[off], srcb.at[slot], sem),
                pltpu.make_async_copy(dst2_hbm.at[off], dstb.at[slot], sem))

    def issue_idx(b, slot, sem):
        for d in idx_copies(b, slot, sem):
            d.start()

    def wait_idx(b, slot, sem):
        for d in idx_copies(b, slot, sem):
            d.wait()

    def run(g_hbm, cs):
        pltpu.sync_copy(g_hbm.at[row], acc_sp.at[row])
        issue_idx(0, 0, sem_i[0])
        plsc.subcore_barrier()

        def superblock(q, carry):
            for half in (0, 1):
                b = q * 2 + half
                wait_idx(b, half, sem_i[half])

                @pl.when(b < NB - 1)
                def _():
                    issue_idx(b + 1, 1 - half, sem_i[1 - half])

                sb = srcb.at[half]
                db = dstb.at[half]
                pltpu.async_copy(g_hbm.at[sb.at[0]], rows.at[0], sem_g0)

                def pair(g, carry2):
                    j0 = g * 2
                    j1 = j0 + 1
                    pltpu.async_copy(g_hbm.at[sb.at[j1]], rows.at[1], sem_g1)
                    pltpu.make_async_copy(
                        g_hbm.at[sb.at[j0]], rows.at[0], sem_g0).wait()
                    pltpu.sync_copy(rows.at[0], acc_sp.at[db.at[j0]], add=True)

                    @pl.when(g < IB // 2 - 1)
                    def _():
                        pltpu.async_copy(
                            g_hbm.at[sb.at[j0 + 2]], rows.at[0], sem_g0)

                    pltpu.make_async_copy(
                        g_hbm.at[sb.at[j1]], rows.at[1], sem_g1).wait()
                    pltpu.sync_copy(rows.at[1], acc_sp.at[db.at[j1]], add=True)
                    return carry2

                lax.fori_loop(0, IB // 2, pair, 0)
            return carry

        lax.fori_loop(0, NB // 2, superblock, 0)
        plsc.subcore_barrier()
        pltpu.sync_copy(acc_sp.at[row], a_hbm.at[row, cs])

    @pl.when(c == 0)
    def _():
        run(g0_hbm, pl.ds(0, HALF))

    @pl.when(c == 1)
    def _():
        run(g1_hbm, pl.ds(HALF, HALF))


@functools.cache
def _get_prop():
    return pl.kernel(
        _prop_body,
        out_type=[
            jax.ShapeDtypeStruct((N, EMB), jnp.float32),
        ],
        mesh=plsc.VectorSubcoreMesh(
            core_axis_name="c", subcore_axis_name="s",
            num_cores=NSC, num_subcores=NSUB),
        scratch_types=[
            pltpu.VMEM_SHARED((N, HALF), jnp.float32),
            pltpu.VMEM((2, IB, PC), jnp.int32),
            pltpu.VMEM((2, IB, PC), jnp.int32),
            pltpu.VMEM((2, PC, HALF), jnp.float32),
            pltpu.SemaphoreType.DMA,
            pltpu.SemaphoreType.DMA,
            pltpu.SemaphoreType.DMA,
            pltpu.SemaphoreType.DMA,
        ],
        compiler_params=pltpu.CompilerParams(use_tc_tiling_on_sc=False),
    )

BLK = 2000
GRID = N // BLK


def _tc0_body(x_ref, w_ref, p_ref, g0_ref, g1_ref, dinv_ref):
    deg = 1.0 + p_ref[:, 0:1] + p_ref[:, 8:9]
    dinv = lax.rsqrt(deg)
    h = jnp.dot(x_ref[...], w_ref[...], **_MM)
    gg = h * dinv
    g0_ref[...] = gg[:, :HALF]
    g1_ref[...] = gg[:, HALF:]
    dinv_ref[...] = dinv


_tc0 = pl.pallas_call(
    _tc0_body,
    grid=(GRID,),
    in_specs=[
        pl.BlockSpec((BLK, IN_DIM), lambda i: (i, 0)),
        pl.BlockSpec((IN_DIM, EMB), lambda i: (0, 0)),
        pl.BlockSpec((BLK, 16), lambda i: (i, 0)),
    ],
    out_specs=[
        pl.BlockSpec((BLK, HALF), lambda i: (i, 0)),
        pl.BlockSpec((BLK, HALF), lambda i: (i, 0)),
        pl.BlockSpec((BLK, 1), lambda i: (i, 0)),
    ],
    out_shape=[
        jax.ShapeDtypeStruct((N, HALF), jnp.float32),
        jax.ShapeDtypeStruct((N, HALF), jnp.float32),
        jax.ShapeDtypeStruct((N, 1), jnp.float32),
    ],
)


def _tc_mid_body(a_ref, dinv_ref, b_ref, w_ref, g0_ref, g1_ref):
    dinv = dinv_ref[...]
    act = jnp.maximum(a_ref[...] * dinv + b_ref[...], 0.0)
    gg = jnp.dot(act, w_ref[...], **_MM) * dinv
    g0_ref[...] = gg[:, :HALF]
    g1_ref[...] = gg[:, HALF:]


_tc_mid = pl.pallas_call(
    _tc_mid_body,
    grid=(GRID,),
    in_specs=[
        pl.BlockSpec((BLK, EMB), lambda i: (i, 0)),
        pl.BlockSpec((BLK, 1), lambda i: (i, 0)),
        pl.BlockSpec((1, EMB), lambda i: (0, 0)),
        pl.BlockSpec((EMB, EMB), lambda i: (0, 0)),
    ],
    out_specs=[
        pl.BlockSpec((BLK, HALF), lambda i: (i, 0)),
        pl.BlockSpec((BLK, HALF), lambda i: (i, 0)),
    ],
    out_shape=[
        jax.ShapeDtypeStruct((N, HALF), jnp.float32),
        jax.ShapeDtypeStruct((N, HALF), jnp.float32),
    ],
)


def _tc3_body(a_ref, dinv_ref, b2_ref, h1w_ref, h1b_ref,
              h2w_ref, h2b_ref, h3w_ref, h3b_ref, out_ref):
    h3 = a_ref[...] * dinv_ref[...] + b2_ref[...]
    t = jnp.maximum(jnp.dot(h3, h1w_ref[...], **_MM) + h1b_ref[...], 0.0)
    t = jnp.maximum(jnp.dot(t, h2w_ref[...], **_MM) + h2b_ref[...], 0.0)
    out_ref[...] = jnp.dot(t, h3w_ref[...], **_MM) + h3b_ref[...]


_tc3 = pl.pallas_call(
    _tc3_body,
    grid=(GRID,),
    in_specs=[
        pl.BlockSpec((BLK, EMB), lambda i: (i, 0)),
        pl.BlockSpec((BLK, 1), lambda i: (i, 0)),
        pl.BlockSpec((1, EMB), lambda i: (0, 0)),
        pl.BlockSpec((EMB, EMB), lambda i: (0, 0)),
        pl.BlockSpec((1, EMB), lambda i: (0, 0)),
        pl.BlockSpec((EMB, EMB), lambda i: (0, 0)),
        pl.BlockSpec((1, EMB), lambda i: (0, 0)),
        pl.BlockSpec((EMB, EMB), lambda i: (0, 0)),
        pl.BlockSpec((1, EMB), lambda i: (0, 0)),
    ],
    out_specs=[pl.BlockSpec((BLK, EMB), lambda i: (i, 0))],
    out_shape=[jax.ShapeDtypeStruct((N, EMB), jnp.float32)],
)


def kernel(x, edge_index, W0, b0, W1, b1, W2, b2, H1w, H1b, H2w, H2b, H3w, H3b):
    src2 = edge_index[0].reshape(PROWS, PC)
    dst2 = edge_index[1].reshape(PROWS, PC)
    ones = jnp.ones((PC, 8), jnp.float32)
    zeros = jnp.zeros((N, 8), jnp.float32)

    _prop = _get_prop()
    (p,) = _get_deg()(dst2, ones, zeros)
    g0, g1, dinv = _tc0(x, W0, p)
    (a,) = _prop(g0, g1, src2, dst2)
    g0, g1 = _tc_mid(a, dinv, b0.reshape(1, EMB), W1)
    (a,) = _prop(g0, g1, src2, dst2)
    g0, g1 = _tc_mid(a, dinv, b1.reshape(1, EMB), W2)
    (a,) = _prop(g0, g1, src2, dst2)
    (out,) = _tc3(a, dinv, b2.reshape(1, EMB),
                  H1w, H1b.reshape(1, EMB), H2w, H2b.reshape(1, EMB),
                  H3w, H3b.reshape(1, EMB))
    return out


# speedup vs baseline: 24.9994x; 1.2175x over previous
import functools

import jax
import jax.numpy as jnp
from jax import lax
from jax.experimental import pallas as pl
from jax.experimental.pallas import tpu as pltpu, tpu_sc as plsc

N = 50000
E = 800000
IN_DIM = 300
EMB = 64
HALF = 32

NSC = 2
NSUB = 16
NPT = N // NSUB

_MM = dict(preferred_element_type=jnp.float32, precision=lax.Precision.HIGHEST)

PC = 100
PROWS = E // PC
DROWS = PROWS // (NSC * NSUB)


def _deg_body(dst2_hbm, ones_hbm, zeros_hbm, p_hbm,
              acc_sp, dsta, ones_v, sem0, sem1):
    c = lax.axis_index("c")
    s = lax.axis_index("s")
    row = pl.ds(s * NPT, NPT)
    pltpu.sync_copy(dst2_hbm.at[pl.ds((c * NSUB + s) * DROWS, DROWS)], dsta)
    pltpu.sync_copy(zeros_hbm.at[row], acc_sp.at[row])
    pltpu.sync_copy(ones_hbm, ones_v)
    plsc.subcore_barrier()

    pltpu.async_copy(ones_v, acc_sp.at[dsta.at[0]], sem0, add=True)
    pltpu.async_copy(ones_v, acc_sp.at[dsta.at[1]], sem1, add=True)

    def pair(g, carry):
        k0 = g * 2
        k1 = k0 + 1
        pltpu.make_async_copy(ones_v, acc_sp.at[dsta.at[k0 - 2]], sem0).wait()
        pltpu.async_copy(ones_v, acc_sp.at[dsta.at[k0]], sem0, add=True)
        pltpu.make_async_copy(ones_v, acc_sp.at[dsta.at[k1 - 2]], sem1).wait()
        pltpu.async_copy(ones_v, acc_sp.at[dsta.at[k1]], sem1, add=True)
        return carry

    lax.fori_loop(1, DROWS // 2, pair, 0)
    pltpu.make_async_copy(ones_v, acc_sp.at[dsta.at[DROWS - 2]], sem0).wait()
    pltpu.make_async_copy(ones_v, acc_sp.at[dsta.at[DROWS - 1]], sem1).wait()
    plsc.subcore_barrier()

    @pl.when(c == 0)
    def _():
        pltpu.sync_copy(acc_sp.at[row], p_hbm.at[row, pl.ds(0, 8)])

    @pl.when(c == 1)
    def _():
        pltpu.sync_copy(acc_sp.at[row], p_hbm.at[row, pl.ds(8, 8)])


@functools.cache
def _get_deg():
    return pl.kernel(
        _deg_body,
        out_type=[
            jax.ShapeDtypeStruct((N, 16), jnp.float32),
        ],
        mesh=plsc.VectorSubcoreMesh(
            core_axis_name="c", subcore_axis_name="s",
            num_cores=NSC, num_subcores=NSUB),
        scratch_types=[
            pltpu.VMEM_SHARED((N, 8), jnp.float32),
            pltpu.VMEM((DROWS, PC), jnp.int32),
            pltpu.VMEM((PC, 8), jnp.float32),
            pltpu.SemaphoreType.DMA,
            pltpu.SemaphoreType.DMA,
        ],
        compiler_params=pltpu.CompilerParams(use_tc_tiling_on_sc=False),
    )

TROWS = PROWS // NSUB
IB = 10
NB = TROWS // IB


def _prop_body(g0_hbm, g1_hbm, src2_hbm, dst2_hbm, a_hbm,
               acc_sp, srcb, dstb, rows,
               sem_i0, sem_i1, sem_g0, sem_g1, sem_g2, sem_g3,
               sem_s0, sem_s1, sem_s2, sem_s3):
    c = lax.axis_index("c")
    s = lax.axis_index("s")
    row = pl.ds(s * NPT, NPT)
    sem_i = (sem_i0, sem_i1)
    sem_g = (sem_g0, sem_g1, sem_g2, sem_g3)
    sem_s = (sem_s0, sem_s1, sem_s2, sem_s3)

    def idx_copies(b, slot, sem):
        off = pl.ds(s * TROWS + b * IB, IB)
        return (pltpu.make_async_copy(src2_hbm.at[off], srcb.at[slot], sem),
                pltpu.make_async_copy(dst2_hbm.at[off], dstb.at[slot], sem))

    def issue_idx(b, slot, sem):
        for d in idx_copies(b, slot, sem):
            d.start()

    def wait_idx(b, slot, sem):
        for d in idx_copies(b, slot, sem):
            d.wait()

    def run(g_hbm, cs):
        pltpu.sync_copy(g_hbm.at[row], acc_sp.at[row])
        issue_idx(0, 0, sem_i[0])
        plsc.subcore_barrier()

        def gather(k_row, slot):
            pltpu.async_copy(g_hbm.at[k_row], rows.at[slot], sem_g[slot])

        def wait_gather(k_row, slot):
            pltpu.make_async_copy(g_hbm.at[k_row], rows.at[slot],
                                  sem_g[slot]).wait()

        def scatter(d_row, slot):
            pltpu.async_copy(rows.at[slot], acc_sp.at[d_row], sem_s[slot],
                             add=True)

        def wait_scatter(d_row, slot):
            pltpu.make_async_copy(rows.at[slot], acc_sp.at[d_row],
                                  sem_s[slot]).wait()

        def superblock(q, carry):
            for half in (0, 1):
                b = q * 2 + half
                wait_idx(b, half, sem_i[half])
                sb = srcb.at[half]
                db = dstb.at[half]

                for j in (0, 1):
                    slot = j

                    @pl.when(b * IB + j >= 4)
                    def _():
                        wait_scatter(acc_sp_dummy_idx(db, j), slot)

                    gather(sb.at[j], slot)

                for j in range(IB):
                    slot = j % 4
                    if j == 2:
                        @pl.when(b < NB - 1)
                        def _():
                            issue_idx(b + 1, 1 - half, sem_i[1 - half])
                    if j < IB - 2:
                        nslot = (j + 2) % 4

                        @pl.when(b * IB + j >= 2)
                        def _():
                            wait_scatter(acc_sp_dummy_idx(db, j), nslot)

                        gather(sb.at[j + 2], nslot)
                    wait_gather(sb.at[j], slot)
                    scatter(db.at[j], slot)
            return carry

        def acc_sp_dummy_idx(db, j):
            return db.at[j]

        lax.fori_loop(0, NB // 2, superblock, 0)
        for slot in range(4):
            pltpu.make_async_copy(rows.at[slot],
                                  acc_sp.at[dstb.at[1].at[IB - 1]],
                                  sem_s[slot]).wait()
        plsc.subcore_barrier()
        pltpu.sync_copy(acc_sp.at[row], a_hbm.at[row, cs])

    @pl.when(c == 0)
    def _():
        run(g0_hbm, pl.ds(0, HALF))

    @pl.when(c == 1)
    def _():
        run(g1_hbm, pl.ds(HALF, HALF))


@functools.cache
def _get_prop():
    return pl.kernel(
        _prop_body,
        out_type=[
            jax.ShapeDtypeStruct((N, EMB), jnp.float32),
        ],
        mesh=plsc.VectorSubcoreMesh(
            core_axis_name="c", subcore_axis_name="s",
            num_cores=NSC, num_subcores=NSUB),
        scratch_types=[
            pltpu.VMEM_SHARED((N, HALF), jnp.float32),
            pltpu.VMEM((2, IB, PC), jnp.int32),
            pltpu.VMEM((2, IB, PC), jnp.int32),
            pltpu.VMEM((4, PC, HALF), jnp.float32),
        ] + [pltpu.SemaphoreType.DMA] * 10,
        compiler_params=pltpu.CompilerParams(use_tc_tiling_on_sc=False),
    )

BLK = 2000
GRID = N // BLK


def _tc0_body(x_ref, w_ref, p_ref, g0_ref, g1_ref, dinv_ref):
    deg = 1.0 + p_ref[:, 0:1] + p_ref[:, 8:9]
    dinv = lax.rsqrt(deg)
    h = jnp.dot(x_ref[...], w_ref[...], **_MM)
    gg = h * dinv
    g0_ref[...] = gg[:, :HALF]
    g1_ref[...] = gg[:, HALF:]
    dinv_ref[...] = dinv


_tc0 = pl.pallas_call(
    _tc0_body,
    grid=(GRID,),
    in_specs=[
        pl.BlockSpec((BLK, IN_DIM), lambda i: (i, 0)),
        pl.BlockSpec((IN_DIM, EMB), lambda i: (0, 0)),
        pl.BlockSpec((BLK, 16), lambda i: (i, 0)),
    ],
    out_specs=[
        pl.BlockSpec((BLK, HALF), lambda i: (i, 0)),
        pl.BlockSpec((BLK, HALF), lambda i: (i, 0)),
        pl.BlockSpec((BLK, 1), lambda i: (i, 0)),
    ],
    out_shape=[
        jax.ShapeDtypeStruct((N, HALF), jnp.float32),
        jax.ShapeDtypeStruct((N, HALF), jnp.float32),
        jax.ShapeDtypeStruct((N, 1), jnp.float32),
    ],
)


def _tc_mid_body(a_ref, dinv_ref, b_ref, w_ref, g0_ref, g1_ref):
    dinv = dinv_ref[...]
    act = jnp.maximum(a_ref[...] * dinv + b_ref[...], 0.0)
    gg = jnp.dot(act, w_ref[...], **_MM) * dinv
    g0_ref[...] = gg[:, :HALF]
    g1_ref[...] = gg[:, HALF:]


_tc_mid = pl.pallas_call(
    _tc_mid_body,
    grid=(GRID,),
    in_specs=[
        pl.BlockSpec((BLK, EMB), lambda i: (i, 0)),
        pl.BlockSpec((BLK, 1), lambda i: (i, 0)),
        pl.BlockSpec((1, EMB), lambda i: (0, 0)),
        pl.BlockSpec((EMB, EMB), lambda i: (0, 0)),
    ],
    out_specs=[
        pl.BlockSpec((BLK, HALF), lambda i: (i, 0)),
        pl.BlockSpec((BLK, HALF), lambda i: (i, 0)),
    ],
    out_shape=[
        jax.ShapeDtypeStruct((N, HALF), jnp.float32),
        jax.ShapeDtypeStruct((N, HALF), jnp.float32),
    ],
)


def _tc3_body(a_ref, dinv_ref, b2_ref, h1w_ref, h1b_ref,
              h2w_ref, h2b_ref, h3w_ref, h3b_ref, out_ref):
    h3 = a_ref[...] * dinv_ref[...] + b2_ref[...]
    t = jnp.maximum(jnp.dot(h3, h1w_ref[...], **_MM) + h1b_ref[...], 0.0)
    t = jnp.maximum(jnp.dot(t, h2w_ref[...], **_MM) + h2b_ref[...], 0.0)
    out_ref[...] = jnp.dot(t, h3w_ref[...], **_MM) + h3b_ref[...]


_tc3 = pl.pallas_call(
    _tc3_body,
    grid=(GRID,),
    in_specs=[
        pl.BlockSpec((BLK, EMB), lambda i: (i, 0)),
        pl.BlockSpec((BLK, 1), lambda i: (i, 0)),
        pl.BlockSpec((1, EMB), lambda i: (0, 0)),
        pl.BlockSpec((EMB, EMB), lambda i: (0, 0)),
        pl.BlockSpec((1, EMB), lambda i: (0, 0)),
        pl.BlockSpec((EMB, EMB), lambda i: (0, 0)),
        pl.BlockSpec((1, EMB), lambda i: (0, 0)),
        pl.BlockSpec((EMB, EMB), lambda i: (0, 0)),
        pl.BlockSpec((1, EMB), lambda i: (0, 0)),
    ],
    out_specs=[pl.BlockSpec((BLK, EMB), lambda i: (i, 0))],
    out_shape=[jax.ShapeDtypeStruct((N, EMB), jnp.float32)],
)


def kernel(x, edge_index, W0, b0, W1, b1, W2, b2, H1w, H1b, H2w, H2b, H3w, H3b):
    src2 = edge_index[0].reshape(PROWS, PC)
    dst2 = edge_index[1].reshape(PROWS, PC)
    ones = jnp.ones((PC, 8), jnp.float32)
    zeros = jnp.zeros((N, 8), jnp.float32)

    _prop = _get_prop()
    (p,) = _get_deg()(dst2, ones, zeros)
    g0, g1, dinv = _tc0(x, W0, p)
    (a,) = _prop(g0, g1, src2, dst2)
    g0, g1 = _tc_mid(a, dinv, b0.reshape(1, EMB), W1)
    (a,) = _prop(g0, g1, src2, dst2)
    g0, g1 = _tc_mid(a, dinv, b1.reshape(1, EMB), W2)
    (a,) = _prop(g0, g1, src2, dst2)
    (out,) = _tc3(a, dinv, b2.reshape(1, EMB),
                  H1w, H1b.reshape(1, EMB), H2w, H2b.reshape(1, EMB),
                  H3w, H3b.reshape(1, EMB))
    return out
